# baseline (speedup 1.0000x reference)
import functools

import jax
import jax.numpy as jnp
from jax import lax
from jax.experimental import pallas as pl
from jax.experimental.pallas import tpu as pltpu
from jax.experimental.pallas import tpu_sc as plsc

_NC = 2
_NS = 16
_NW = _NC * _NS


def _sc_gather_pairs(up, bp, user_id, isbn_id):
    B = user_id.shape[0]
    W = up.shape[1]
    bpw = B // _NW
    CHR = 256
    mesh = plsc.VectorSubcoreMesh(core_axis_name="c", subcore_axis_name="s")
    out_ty = jax.ShapeDtypeStruct((B, W), up.dtype)
    L = 16

    @functools.partial(
        pl.kernel,
        mesh=mesh,
        out_type=(out_ty, out_ty),
        scratch_types=[
            pltpu.VMEM((bpw,), jnp.int32),
            pltpu.VMEM((CHR, W), jnp.float32),
        ],
    )
    def k(ut_hbm, bt_hbm, uid_hbm, bid_hbm, uout_hbm, bout_hbm,
          idx_v, rows_v):
        wid = lax.axis_index("s") * _NC + lax.axis_index("c")
        base = wid * bpw

        def gather_to(table_hbm, id_hbm, out_hbm):
            half = table_hbm.shape[0]
            pltpu.sync_copy(id_hbm.at[pl.ds(base, bpw)], idx_v)

            @pl.loop(0, bpw, step=L)
            def _(j):
                idx_v[pl.ds(j, L)] = lax.rem(idx_v[pl.ds(j, L)], half)

            @pl.loop(0, bpw, step=CHR)
            def _(c):
                pltpu.sync_copy(table_hbm.at[idx_v.at[pl.ds(c, CHR)]], rows_v)
                pltpu.sync_copy(rows_v, out_hbm.at[pl.ds(base + c, CHR)])

        gather_to(ut_hbm, uid_hbm, uout_hbm)
        gather_to(bt_hbm, bid_hbm, bout_hbm)

    return k(up, bp, user_id, isbn_id)


def _mlp_body(u_ref, b_ref, pu_ref, pb_ref, w1a_ref, w1b_ref, b1_ref,
              w2_ref, b2_ref, w3t_ref, b3_ref, o_ref):
    d = w1a_ref.shape[0]
    upair = u_ref[...]
    bpair = b_ref[...]
    u = jnp.where(pu_ref[...] > 0.5, upair[:, d:], upair[:, :d])
    b = jnp.where(pb_ref[...] > 0.5, bpair[:, d:], bpair[:, :d])
    h = (
        jnp.dot(u, w1a_ref[...], preferred_element_type=jnp.float32)
        + jnp.dot(b, w1b_ref[...], preferred_element_type=jnp.float32)
        + b1_ref[...]
    )
    h = jnp.maximum(h, 0.0)
    h = jnp.dot(h, w2_ref[...], preferred_element_type=jnp.float32) + b2_ref[...]
    h = jnp.maximum(h, 0.0)
    o_ref[...] = (
        jnp.sum(h * w3t_ref[...], axis=1, keepdims=True) + b3_ref[...]
    )


def _tc_mlp(u_pack, b_pack, pu, pb, W1, b1, W2, b2, W3, b3, block_b=2048):
    B = u_pack.shape[0]
    D = W1.shape[0] // 2
    H1 = W1.shape[1]
    H2 = W2.shape[1]
    w1a = W1[:D]
    w1b = W1[D:]
    b1r = b1.reshape(1, H1)
    b2r = b2.reshape(1, H2)
    w3t = W3.reshape(1, H2)
    b3r = b3.reshape(1, 1)
    grid = (B // block_b,)

    def full(shape):
        return pl.BlockSpec(shape, lambda i: (0, 0))

    return pl.pallas_call(
        _mlp_body,
        grid=grid,
        in_specs=[
            pl.BlockSpec((block_b, 2 * D), lambda i: (i, 0)),
            pl.BlockSpec((block_b, 2 * D), lambda i: (i, 0)),
            pl.BlockSpec((block_b, 1), lambda i: (i, 0)),
            pl.BlockSpec((block_b, 1), lambda i: (i, 0)),
            full((D, H1)),
            full((D, H1)),
            full((1, H1)),
            full((H1, H2)),
            full((1, H2)),
            full((1, H2)),
            full((1, 1)),
        ],
        out_specs=pl.BlockSpec((block_b, 1), lambda i: (i, 0)),
        out_shape=jax.ShapeDtypeStruct((B, 1), jnp.float32),
    )(u_pack, b_pack, pu, pb, w1a, w1b, b1r, W2, b2r, w3t, b3r)


_PACK_C = 512


def _pack_body(lo_ref, hi_ref, o_ref):
    d = lo_ref.shape[0]
    o_ref[:, :d] = lo_ref[...].T
    o_ref[:, d:] = hi_ref[...].T


def _pack(table, half):
    V, D = table.shape
    tt = table.T
    n = half // _PACK_C
    return pl.pallas_call(
        _pack_body,
        grid=(n,),
        in_specs=[
            pl.BlockSpec((D, _PACK_C), lambda i: (0, i)),
            pl.BlockSpec((D, _PACK_C), lambda i, n_=n: (0, i + n_)),
        ],
        out_specs=pl.BlockSpec((_PACK_C, 2 * D), lambda i: (i, 0)),
        out_shape=jax.ShapeDtypeStruct((half, 2 * D), table.dtype),
    )(tt, tt)


def _half_rows(v):
    return ((v + 2 * _PACK_C - 1) // (2 * _PACK_C)) * _PACK_C


def kernel(user_id, isbn_id, user_table, book_table, W1, b1, W2, b2, W3, b3):
    hu = _half_rows(user_table.shape[0])
    hb = _half_rows(book_table.shape[0])
    up = _pack(user_table, hu)
    bp = _pack(book_table, hb)
    uid = user_id.astype(jnp.int32)
    bid = isbn_id.astype(jnp.int32)
    u_pack, b_pack = _sc_gather_pairs(up, bp, uid, bid)
    B = uid.shape[0]
    pu = (uid >= hu).astype(jnp.float32).reshape(B, 1)
    pb = (bid >= hb).astype(jnp.float32).reshape(B, 1)
    return _tc_mlp(u_pack, b_pack, pu, pb, W1, b1, W2, b2, W3, b3)


# speedup vs baseline: 1.9803x; 1.9803x over previous
import functools

import jax
import jax.numpy as jnp
from jax import lax
from jax.experimental import pallas as pl
from jax.experimental.pallas import tpu as pltpu
from jax.experimental.pallas import tpu_sc as plsc

_NC = 2
_NS = 16
_NW = _NC * _NS


def _sc_gather_pair(user_table, book_table, user_id, isbn_id):
    B = user_id.shape[0]
    D = user_table.shape[1]
    bpw = B // _NW
    mesh = plsc.VectorSubcoreMesh(core_axis_name="c", subcore_axis_name="s")
    row_ty = jax.ShapeDtypeStruct((B, D), user_table.dtype)

    CH = 64
    G = 8

    @functools.partial(
        pl.kernel,
        mesh=mesh,
        out_type=(row_ty, row_ty),
        scratch_types=[
            pltpu.VMEM((bpw,), jnp.int32),
            pltpu.VMEM((bpw,), jnp.int32),
            pltpu.VMEM((bpw // 2, D), jnp.float32),
            pltpu.VMEM((bpw // 2, D), jnp.float32),
            pltpu.SemaphoreType.DMA,
            pltpu.SemaphoreType.DMA,
        ],
    )
    def k(ut_hbm, bt_hbm, uid_hbm, bid_hbm, uout_hbm, bout_hbm,
          uidx_v, bidx_v, urows_v, brows_v, usem, bsem):
        wid = lax.axis_index("s") * _NC + lax.axis_index("c")
        base = wid * bpw
        L = 16

        half = bpw // 2
        pltpu.sync_copy(uid_hbm.at[pl.ds(base, bpw)], uidx_v)
        pltpu.sync_copy(bid_hbm.at[pl.ds(base, bpw)], bidx_v)

        @pl.loop(0, bpw, step=half)
        def _(c):
            @pl.loop(0, half, step=L)
            def _(j):
                vu = uidx_v[pl.ds(c + j, L)]
                vb = bidx_v[pl.ds(c + j, L)]
                for t in range(L):
                    pltpu.async_copy(
                        ut_hbm.at[vu[t]], urows_v.at[j + t], usem)
                    pltpu.async_copy(
                        bt_hbm.at[vb[t]], brows_v.at[j + t], bsem)

            @pl.loop(0, half)
            def _(j):
                pltpu.make_async_copy(
                    ut_hbm.at[0], urows_v.at[0], usem).wait()
                pltpu.make_async_copy(
                    bt_hbm.at[0], brows_v.at[0], bsem).wait()

            pltpu.sync_copy(urows_v, uout_hbm.at[pl.ds(base + c, half)])
            pltpu.sync_copy(brows_v, bout_hbm.at[pl.ds(base + c, half)])

    return k(user_table, book_table, user_id, isbn_id)


def _mlp_body(u_ref, b_ref, w1_ref, b1_ref, w2_ref, b2_ref,
              w3t_ref, b3_ref, o_ref):
    d = u_ref.shape[1]
    h = (
        jnp.dot(u_ref[...], w1_ref[:d], preferred_element_type=jnp.float32)
        + jnp.dot(b_ref[...], w1_ref[d:], preferred_element_type=jnp.float32)
        + b1_ref[...]
    )
    h = jnp.maximum(h, 0.0)
    h = jnp.dot(h, w2_ref[...], preferred_element_type=jnp.float32) + b2_ref[...]
    h = jnp.maximum(h, 0.0)
    o_ref[...] = (
        lax.dot_general(w3t_ref[...], h, (((1,), (1,)), ((), ())),
                        preferred_element_type=jnp.float32)
        + b3_ref[...]
    )


def _tc_mlp(u, b, W1, b1, W2, b2, W3, b3, block_b=2048):
    B = u.shape[0]
    D = W1.shape[0] // 2
    H1 = W1.shape[1]
    H2 = W2.shape[1]
    b1r = b1.reshape(1, H1)
    b2r = b2.reshape(1, H2)
    w3t = W3.reshape(1, H2)
    b3r = b3.reshape(1, 1)
    grid = (B // block_b,)

    def full(shape):
        return pl.BlockSpec(shape, lambda i: (0, 0))

    out = pl.pallas_call(
        _mlp_body,
        grid=grid,
        in_specs=[
            pl.BlockSpec((block_b, D), lambda i: (i, 0)),
            pl.BlockSpec((block_b, D), lambda i: (i, 0)),
            full((2 * D, H1)),
            full((1, H1)),
            full((H1, H2)),
            full((1, H2)),
            full((1, H2)),
            full((1, 1)),
        ],
        out_specs=pl.BlockSpec((1, block_b), lambda i: (0, i)),
        out_shape=jax.ShapeDtypeStruct((1, B), jnp.float32),
    )(u, b, W1, b1r, W2, b2r, w3t, b3r)
    return out.reshape(B, 1)


def kernel(user_id, isbn_id, user_table, book_table, W1, b1, W2, b2, W3, b3):
    u_rows, b_rows = _sc_gather_pair(
        user_table, book_table, user_id, isbn_id)
    return _tc_mlp(u_rows, b_rows, W1, b1, W2, b2, W3, b3)


# speedup vs baseline: 1.9852x; 1.0025x over previous
import functools

import jax
import jax.numpy as jnp
from jax import lax
from jax.experimental import pallas as pl
from jax.experimental.pallas import tpu as pltpu
from jax.experimental.pallas import tpu_sc as plsc

_NC = 2
_NS = 16
_NW = _NC * _NS


def _sc_gather_pair(user_table, book_table, user_id, isbn_id):
    B = user_id.shape[0]
    D = user_table.shape[1]
    bpw = B // _NW
    mesh = plsc.VectorSubcoreMesh(core_axis_name="c", subcore_axis_name="s")
    row_ty = jax.ShapeDtypeStruct((B, D), user_table.dtype)

    CH = 64
    G = 8

    @functools.partial(
        pl.kernel,
        mesh=mesh,
        out_type=(row_ty, row_ty),
        scratch_types=[
            pltpu.VMEM((bpw,), jnp.int32),
            pltpu.VMEM((bpw,), jnp.int32),
            pltpu.VMEM((bpw // 2, D), jnp.float32),
            pltpu.VMEM((bpw // 2, D), jnp.float32),
            pltpu.SemaphoreType.DMA,
            pltpu.SemaphoreType.DMA,
        ],
    )
    def k(ut_hbm, bt_hbm, uid_hbm, bid_hbm, uout_hbm, bout_hbm,
          uidx_v, bidx_v, urows_v, brows_v, usem, bsem):
        wid = lax.axis_index("s") * _NC + lax.axis_index("c")
        base = wid * bpw
        L = 16

        half = bpw // 2
        pltpu.sync_copy(uid_hbm.at[pl.ds(base, bpw)], uidx_v)
        pltpu.sync_copy(bid_hbm.at[pl.ds(base, bpw)], bidx_v)

        @pl.loop(0, bpw, step=half)
        def _(c):
            @pl.loop(0, half, step=L)
            def _(j):
                vu = uidx_v[pl.ds(c + j, L)]
                vb = bidx_v[pl.ds(c + j, L)]
                for t in range(L):
                    pltpu.async_copy(
                        ut_hbm.at[vu[t]], urows_v.at[j + t], usem)
                    pltpu.async_copy(
                        bt_hbm.at[vb[t]], brows_v.at[j + t], bsem)

            @pl.loop(0, half)
            def _(j):
                pltpu.make_async_copy(
                    ut_hbm.at[0], urows_v.at[0], usem).wait()
                pltpu.make_async_copy(
                    bt_hbm.at[0], brows_v.at[0], bsem).wait()

            pltpu.sync_copy(urows_v, uout_hbm.at[pl.ds(base + c, half)])
            pltpu.sync_copy(brows_v, bout_hbm.at[pl.ds(base + c, half)])

    return k(user_table, book_table, user_id, isbn_id)


def _mlp_body(u_ref, b_ref, w1_ref, b1_ref, w2_ref, b2_ref,
              w3t_ref, b3_ref, o_ref):
    d = u_ref.shape[1]
    h = (
        jnp.dot(u_ref[...], w1_ref[:d], preferred_element_type=jnp.float32)
        + jnp.dot(b_ref[...], w1_ref[d:], preferred_element_type=jnp.float32)
        + b1_ref[...]
    )
    h = jnp.maximum(h, 0.0)
    h = jnp.dot(h, w2_ref[...], preferred_element_type=jnp.float32) + b2_ref[...]
    h = jnp.maximum(h, 0.0)
    o_ref[...] = (
        lax.dot_general(w3t_ref[...], h, (((1,), (1,)), ((), ())),
                        preferred_element_type=jnp.float32)
        + b3_ref[...]
    )


def _tc_mlp(u, b, W1, b1, W2, b2, W3, b3, block_b=2048):
    B = u.shape[0]
    D = W1.shape[0] // 2
    H1 = W1.shape[1]
    H2 = W2.shape[1]
    b1r = b1.reshape(1, H1)
    b2r = b2.reshape(1, H2)
    w3t = W3.reshape(1, H2)
    b3r = b3.reshape(1, 1)
    grid = (B // block_b,)

    def full(shape):
        return pl.BlockSpec(shape, lambda i: (0, 0))

    out = pl.pallas_call(
        _mlp_body,
        grid=grid,
        in_specs=[
            pl.BlockSpec((block_b, D), lambda i: (i, 0)),
            pl.BlockSpec((block_b, D), lambda i: (i, 0)),
            full((2 * D, H1)),
            full((1, H1)),
            full((H1, H2)),
            full((1, H2)),
            full((1, H2)),
            full((1, 1)),
        ],
        out_specs=pl.BlockSpec((1, block_b), lambda i: (0, i)),
        out_shape=jax.ShapeDtypeStruct((1, B), jnp.float32),
    )(u, b, W1, b1r, W2, b2r, w3t, b3r)
    return out.reshape(B, 1)


def kernel(user_id, isbn_id, user_table, book_table, W1, b1, W2, b2, W3, b3):
    u_rows, b_rows = _sc_gather_pair(
        user_table, book_table,
        user_id.astype(jnp.int32), isbn_id.astype(jnp.int32))
    return _tc_mlp(u_rows, b_rows, W1, b1, W2, b2, W3, b3)
